# baseline (speedup 1.0000x reference)
import functools

import jax
import jax.numpy as jnp
from jax import lax
from jax.experimental import pallas as pl
from jax.experimental.pallas import tpu as pltpu

K = 32


def _sortable_key(f):
    b = lax.bitcast_convert_type(f, jnp.int32)
    return jnp.where(b < 0, jnp.bitwise_xor(b, jnp.int32(0x7FFFFFFF)), b)


def _key_to_f32(k):
    b = jnp.where(k < 0, jnp.bitwise_xor(k, jnp.int32(0x7FFFFFFF)), k)
    return lax.bitcast_convert_type(b, jnp.float32)


def _mid(lo, hi):
    return (lo >> 1) + (hi >> 1) + (lo & hi & 1)


def _kernel_body(num_g, gt, temp_ref, x_ref, w_ref, out_ref):
    g = pl.program_id(1)
    logits = lax.dot_general(
        x_ref[...], w_ref[...], (((1,), (1,)), ((), ())),
        preferred_element_type=jnp.float32,
    ) / temp_ref[0]
    out_ref[:, pl.ds(g * gt, gt)] = logits

    @pl.when(g == num_g - 1)
    def _finalize():
        rb = out_ref.shape[0]
        gfull = out_ref.shape[1]
        strip = min(128, rb)
        cw = gfull // K
        for r0 in range(0, rb, strip):
            l = out_ref[pl.ds(r0, strip), :]
            m = jnp.max(l, axis=1, keepdims=True)
            s = m
            for j in range(K):
                s = jnp.minimum(
                    s, jnp.max(l[:, j * cw:(j + 1) * cw], axis=1,
                               keepdims=True))
            lo0 = _sortable_key(s)
            hi0 = _sortable_key(m) + 1

            def _gap(lo, hi):
                return lax.bitcast_convert_type(hi - lo, jnp.uint32)

            def cond(st):
                lo, hi, it = st
                return jnp.logical_and(it < 20, jnp.any(_gap(lo, hi) > 1))

            def body(st):
                lo, hi, it = st
                m2 = _mid(lo, hi)
                m1 = _mid(lo, m2)
                f1 = _key_to_f32(m1)
                f2 = _key_to_f32(m2)
                r = (jnp.where(l >= f1, jnp.int32(1), jnp.int32(0))
                     + jnp.where(l >= f2, jnp.int32(1 << 14), jnp.int32(0)))
                sums = jnp.sum(r, axis=1, keepdims=True)
                c2 = sums >> 14
                c1 = sums & jnp.int32(0x3FFF)
                active = _gap(lo, hi) > 1
                ge2 = c2 >= K
                ge1 = c1 >= K
                eq2 = c2 == K
                eq1 = jnp.logical_and(~ge2, c1 == K)
                nlo = jnp.where(ge2, m2, jnp.where(ge1, m1, lo))
                nhi = jnp.where(ge2, hi, jnp.where(ge1, m2, m1))
                nhi = jnp.where(eq2 | eq1, nlo + 1, nhi)
                lo = jnp.where(active, nlo, lo)
                hi = jnp.where(active, nhi, hi)
                return lo, hi, it + 1

            lo, _, _ = lax.while_loop(cond, body, (lo0, hi0, jnp.int32(0)))
            tf = _key_to_f32(lo)
            e = jnp.where(l >= tf, jnp.exp(l - m), jnp.float32(0.0))
            z = jnp.sum(e, axis=1, keepdims=True)
            out_ref[pl.ds(r0, strip), :] = e / z


def _topk_softmax(x2d, w, temp, rb, gt):
    m, d = x2d.shape
    g = w.shape[0]
    num_g = g // gt
    grid = (m // rb, num_g)
    return pl.pallas_call(
        functools.partial(_kernel_body, num_g, gt),
        grid=grid,
        in_specs=[
            pl.BlockSpec(memory_space=pltpu.SMEM),
            pl.BlockSpec((rb, d), lambda i, j: (i, 0)),
            pl.BlockSpec((gt, d), lambda i, j: (j, 0)),
        ],
        out_specs=pl.BlockSpec((rb, g), lambda i, j: (i, 0)),
        out_shape=jax.ShapeDtypeStruct((m, g), jnp.float32),
        compiler_params=pltpu.CompilerParams(
            dimension_semantics=("arbitrary", "arbitrary"),
            vmem_limit_bytes=100 * 1024 * 1024,
        ),
    )(temp, x2d, w)


@jax.jit
def kernel(x, W, temperature):
    b, t, d = x.shape
    g = W.shape[0]
    temp = jnp.maximum(temperature, 0.1).reshape(1)
    out = _topk_softmax(x.reshape(b * t, d), W, temp, rb=512, gt=256)
    return out.reshape(b, t, g)


# speedup vs baseline: 1.0931x; 1.0931x over previous
import functools

import jax
import jax.numpy as jnp
from jax import lax
from jax.experimental import pallas as pl
from jax.experimental.pallas import tpu as pltpu

K = 32


def _sortable_key(f):
    b = lax.bitcast_convert_type(f, jnp.int32)
    return jnp.where(b < 0, jnp.bitwise_xor(b, jnp.int32(0x7FFFFFFF)), b)


def _key_to_f32(k):
    b = jnp.where(k < 0, jnp.bitwise_xor(k, jnp.int32(0x7FFFFFFF)), k)
    return lax.bitcast_convert_type(b, jnp.float32)


def _mid(lo, hi):
    return (lo >> 1) + (hi >> 1) + (lo & hi & 1)


def _kernel_body(num_g, gt, temp_ref, x_ref, w_ref, out_ref):
    g = pl.program_id(1)
    logits = lax.dot_general(
        x_ref[...], w_ref[...], (((1,), (1,)), ((), ())),
        preferred_element_type=jnp.float32,
    ) / temp_ref[0]
    out_ref[:, pl.ds(g * gt, gt)] = logits

    @pl.when(g == num_g - 1)
    def _finalize():
        rb = out_ref.shape[0]
        gfull = out_ref.shape[1]
        strip = min(128, rb)
        cw = gfull // K
        for r0 in range(0, rb, strip):
            l = out_ref[pl.ds(r0, strip), :]
            m = jnp.max(l, axis=1, keepdims=True)
            s = m
            for j in range(K):
                s = jnp.minimum(
                    s, jnp.max(l[:, j * cw:(j + 1) * cw], axis=1,
                               keepdims=True))
            lo0 = _sortable_key(s)
            hi0 = _sortable_key(m) + 1

            def _gap(lo, hi):
                return lax.bitcast_convert_type(hi - lo, jnp.uint32)

            def cond(st):
                lo, hi, it = st
                return jnp.logical_and(it < 34, jnp.any(_gap(lo, hi) > 1))

            def body(st):
                lo, hi, it = st
                mid = _mid(lo, hi)
                fm = _key_to_f32(mid)
                cnt = jnp.sum((l >= fm).astype(jnp.int32), axis=1,
                              keepdims=True)
                active = _gap(lo, hi) > 1
                ge = cnt >= K
                eq = cnt == K
                nlo = jnp.where(ge, mid, lo)
                nhi = jnp.where(ge, hi, mid)
                nhi = jnp.where(eq, nlo + 1, nhi)
                lo = jnp.where(active, nlo, lo)
                hi = jnp.where(active, nhi, hi)
                return lo, hi, it + 1

            lo, _, _ = lax.while_loop(cond, body, (lo0, hi0, jnp.int32(0)))
            tf = _key_to_f32(lo)
            e = jnp.where(l >= tf, jnp.exp(l - m), jnp.float32(0.0))
            z = jnp.sum(e, axis=1, keepdims=True)
            out_ref[pl.ds(r0, strip), :] = e / z


def _topk_softmax(x2d, w, temp, rb, gt):
    m, d = x2d.shape
    g = w.shape[0]
    num_g = g // gt
    grid = (m // rb, num_g)
    return pl.pallas_call(
        functools.partial(_kernel_body, num_g, gt),
        grid=grid,
        in_specs=[
            pl.BlockSpec(memory_space=pltpu.SMEM),
            pl.BlockSpec((rb, d), lambda i, j: (i, 0)),
            pl.BlockSpec((gt, d), lambda i, j: (j, 0)),
        ],
        out_specs=pl.BlockSpec((rb, g), lambda i, j: (i, 0)),
        out_shape=jax.ShapeDtypeStruct((m, g), jnp.float32),
        compiler_params=pltpu.CompilerParams(
            dimension_semantics=("arbitrary", "arbitrary"),
            vmem_limit_bytes=100 * 1024 * 1024,
        ),
    )(temp, x2d, w)


@jax.jit
def kernel(x, W, temperature):
    b, t, d = x.shape
    g = W.shape[0]
    temp = jnp.maximum(temperature, 0.1).reshape(1)
    out = _topk_softmax(x.reshape(b * t, d), W, temp, rb=512, gt=256)
    return out.reshape(b, t, g)
